# baseline (speedup 1.0000x reference)
import functools

import jax
import jax.numpy as jnp
from jax import lax
from jax.experimental import pallas as pl
from jax.experimental.pallas import tpu as pltpu
from jax.experimental.pallas import tpu_sc as plsc

NNODES = 10000
RANK = 16
WID2 = 2 * RANK
NEDGES = 320000
NBATCH = 8192
EMB = 128

NC = 2
NS = 16
NW = NC * NS

GRP = 128
EPT = 20480
NGRP = EPT // GRP
EPAD = EPT * NS
DUMMY = NNODES
NROWS = 10112
RPT = NROWS // NS

BPT = NBATCH // NW
BGRP = BPT // GRP

_MESH1 = plsc.VectorSubcoreMesh(
    core_axis_name="c", subcore_axis_name="s", num_cores=1, num_subcores=NS
)
_MESH2 = plsc.VectorSubcoreMesh(
    core_axis_name="c", subcore_axis_name="s", num_cores=NC, num_subcores=NS
)
_SC_PARAMS = pltpu.CompilerParams(use_tc_tiling_on_sc=False)

NBUF = 8
NCHUNK = NGRP // NBUF


def _scatter_body(up_hbm, srcg, dstg, zeros_hbm,
                  accdeg,
                  acc_sh, sidx, didx, rows, gsem, ssem):
    sid = lax.axis_index("s")
    r0 = sid * RPT

    pltpu.sync_copy(zeros_hbm.at[pl.ds(r0, RPT)], acc_sh.at[pl.ds(r0, RPT)])

    pltpu.sync_copy(srcg.at[pl.ds(sid * NGRP, NGRP)], sidx)
    pltpu.sync_copy(dstg.at[pl.ds(sid * NGRP, NGRP)], didx)

    plsc.subcore_barrier()

    def body(c, carry):
        j0 = c * NBUF
        gds = [pltpu.async_copy(up_hbm.at[sidx.at[j0 + b]], rows.at[b], gsem)
               for b in range(NBUF)]
        for d in gds:
            d.wait()
        sds = [pltpu.async_copy(rows.at[b], acc_sh.at[didx.at[j0 + b]], ssem,
                                add=True)
               for b in range(NBUF)]
        for d in sds:
            d.wait()
        return carry

    lax.fori_loop(0, NCHUNK, body, 0)

    plsc.subcore_barrier()

    pltpu.sync_copy(acc_sh.at[pl.ds(r0, RPT)], accdeg.at[pl.ds(r0, RPT)])


_scatter_kernel = functools.partial(
    pl.kernel,
    out_type=jax.ShapeDtypeStruct((NROWS, WID2), jnp.float32),
    mesh=_MESH1,
    compiler_params=_SC_PARAMS,
    scratch_types=[
        pltpu.VMEM_SHARED((NROWS, WID2), jnp.float32),
        pltpu.VMEM((NGRP, GRP), jnp.int32),
        pltpu.VMEM((NGRP, GRP), jnp.int32),
        pltpu.VMEM((NBUF, GRP, WID2), jnp.float32),
        pltpu.SemaphoreType.DMA,
        pltpu.SemaphoreType.DMA,
    ],
)(_scatter_body)


def _mean_gather_body(accdeg, batchg, outr, bidx, ab, ov, sem):
    cid = lax.axis_index("c")
    sid = lax.axis_index("s")
    wid = cid * NS + sid

    pltpu.sync_copy(batchg.at[pl.ds(wid * BGRP, BGRP)], bidx)
    ds = []
    for j in range(BGRP):
        sl = pl.ds(j * GRP, GRP)
        ds.append(pltpu.async_copy(accdeg.at[bidx.at[j]], ab.at[sl], sem))
    for d in ds:
        d.wait()

    def body(i, carry):
        a = ab[i, pl.ds(0, RANK)]
        d = ab[i, pl.ds(RANK, RANK)]
        ov[i] = a / jnp.maximum(d, 1.0)
        return carry

    lax.fori_loop(0, BPT, body, 0)

    pltpu.sync_copy(ov, outr.at[pl.ds(wid * BPT, BPT)])


_mean_gather_kernel = functools.partial(
    pl.kernel,
    out_type=jax.ShapeDtypeStruct((NBATCH, RANK), jnp.float32),
    mesh=_MESH2,
    compiler_params=_SC_PARAMS,
    scratch_types=[
        pltpu.VMEM((BGRP, GRP), jnp.int32),
        pltpu.VMEM((BPT, WID2), jnp.float32),
        pltpu.VMEM((BPT, RANK), jnp.float32),
        pltpu.SemaphoreType.DMA,
    ],
)(_mean_gather_body)


def _mm_body(x_ref, v_ref, o_ref):
    o_ref[...] = jnp.dot(x_ref[...], v_ref[...],
                         preferred_element_type=jnp.float32)


def _expand(x, v):
    blk = 1024
    return pl.pallas_call(
        _mm_body,
        grid=(NBATCH // blk,),
        in_specs=[
            pl.BlockSpec((blk, RANK), lambda i: (i, 0)),
            pl.BlockSpec((RANK, EMB), lambda i: (0, 0)),
        ],
        out_specs=pl.BlockSpec((blk, EMB), lambda i: (i, 0)),
        out_shape=jax.ShapeDtypeStruct((NBATCH, EMB), jnp.float32),
    )(x, v)


def kernel(U, V, edge_index, batch):
    pad = EPAD - NEDGES
    src = jnp.concatenate([edge_index[0], jnp.zeros((pad,), jnp.int32)])
    dst = jnp.concatenate(
        [edge_index[1], jnp.full((pad,), DUMMY, jnp.int32)])
    srcg = src.reshape(EPAD // GRP, GRP)
    dstg = dst.reshape(EPAD // GRP, GRP)
    up = jnp.concatenate(
        [U, jnp.ones((NNODES, RANK), jnp.float32)], axis=1)
    zeros = jnp.zeros((NROWS, WID2), jnp.float32)

    accdeg = _scatter_kernel(up, srcg, dstg, zeros)
    outr = _mean_gather_kernel(accdeg, batch.reshape(NBATCH // GRP, GRP))
    return _expand(outr, V)


# speedup vs baseline: 1.4832x; 1.4832x over previous
import functools

import jax
import jax.numpy as jnp
from jax import lax
from jax.experimental import pallas as pl
from jax.experimental.pallas import tpu as pltpu
from jax.experimental.pallas import tpu_sc as plsc

NNODES = 10000
RANK = 16
WID2 = 2 * RANK
NEDGES = 320000
NBATCH = 8192
EMB = 128

NC = 2
NS = 16
NW = NC * NS

GRP = 128
EPT = 20480
NGRP = EPT // GRP
EPAD = EPT * NS
DUMMY = NNODES
NROWS = 10112
RPT = NROWS // NS

BPT = NBATCH // NW
BGRP = BPT // GRP

_MESH1 = plsc.VectorSubcoreMesh(
    core_axis_name="c", subcore_axis_name="s", num_cores=1, num_subcores=NS
)
_MESH2 = plsc.VectorSubcoreMesh(
    core_axis_name="c", subcore_axis_name="s", num_cores=NC, num_subcores=NS
)
_SC_PARAMS = pltpu.CompilerParams(use_tc_tiling_on_sc=False)

NBUF = 8
NCHUNK = NGRP // NBUF


def _scatter_body(u_hbm, srcg, dstg, zeros_hbm, ones_hbm,
                  acc, deg,
                  acc_sh, deg_sh, sidx, didx, rows, ones_v,
                  gsem, ssem, dsem):
    sid = lax.axis_index("s")
    r0 = sid * RPT

    pltpu.sync_copy(zeros_hbm.at[pl.ds(r0, RPT)], acc_sh.at[pl.ds(r0, RPT)])
    pltpu.sync_copy(zeros_hbm.at[pl.ds(r0, RPT)], deg_sh.at[pl.ds(r0, RPT)])

    pltpu.sync_copy(srcg.at[pl.ds(sid * NGRP, NGRP)], sidx)
    pltpu.sync_copy(dstg.at[pl.ds(sid * NGRP, NGRP)], didx)
    pltpu.sync_copy(ones_hbm, ones_v)

    plsc.subcore_barrier()

    def gfire(j0, h):
        return [pltpu.async_copy(u_hbm.at[sidx.at[j0 + b]],
                                 rows.at[h * NBUF + b], gsem)
                for b in range(NBUF)]

    def dfire(j0):
        return [pltpu.async_copy(ones_v, deg_sh.at[didx.at[j0 + b]], dsem,
                                 add=True)
                for b in range(NBUF)]

    def sfire(j0, h):
        return [pltpu.async_copy(rows.at[h * NBUF + b],
                                 acc_sh.at[didx.at[j0 + b]], ssem, add=True)
                for b in range(NBUF)]

    def body(k, carry):
        j0 = 2 * k * NBUF
        j1 = j0 + NBUF
        gA = gfire(j0, 0)
        dA = dfire(j0)
        for d in gA:
            d.wait()
        sA = sfire(j0, 0)
        gB = gfire(j1, 1)
        dB = dfire(j1)
        for d in sA + dA + gB:
            d.wait()
        sB = sfire(j1, 1)
        for d in sB + dB:
            d.wait()
        return carry

    lax.fori_loop(0, NCHUNK // 2, body, 0)

    plsc.subcore_barrier()

    pltpu.sync_copy(acc_sh.at[pl.ds(r0, RPT)], acc.at[pl.ds(r0, RPT)])
    pltpu.sync_copy(deg_sh.at[pl.ds(r0, RPT)], deg.at[pl.ds(r0, RPT)])


_scatter_kernel = functools.partial(
    pl.kernel,
    out_type=[jax.ShapeDtypeStruct((NROWS, RANK), jnp.float32)] * 2,
    mesh=_MESH1,
    compiler_params=_SC_PARAMS,
    scratch_types=[
        pltpu.VMEM_SHARED((NROWS, RANK), jnp.float32),
        pltpu.VMEM_SHARED((NROWS, RANK), jnp.float32),
        pltpu.VMEM((NGRP, GRP), jnp.int32),
        pltpu.VMEM((NGRP, GRP), jnp.int32),
        pltpu.VMEM((2 * NBUF, GRP, RANK), jnp.float32),
        pltpu.VMEM((GRP, RANK), jnp.float32),
        pltpu.SemaphoreType.DMA,
        pltpu.SemaphoreType.DMA,
        pltpu.SemaphoreType.DMA,
    ],
)(_scatter_body)


def _mean_gather_body(acc, deg, batchg, outr, bidx, a0, d0, ov, sem):
    cid = lax.axis_index("c")
    sid = lax.axis_index("s")
    wid = cid * NS + sid

    pltpu.sync_copy(batchg.at[pl.ds(wid * BGRP, BGRP)], bidx)
    ds = []
    for j in range(BGRP):
        sl = pl.ds(j * GRP, GRP)
        ds.append(pltpu.async_copy(acc.at[bidx.at[j]], a0.at[sl], sem))
        ds.append(pltpu.async_copy(deg.at[bidx.at[j]], d0.at[sl], sem))
    for d in ds:
        d.wait()

    def body(i, carry):
        ov[i] = a0[i] / jnp.maximum(d0[i], 1.0)
        return carry

    lax.fori_loop(0, BPT, body, 0)

    pltpu.sync_copy(ov, outr.at[pl.ds(wid * BPT, BPT)])


_mean_gather_kernel = functools.partial(
    pl.kernel,
    out_type=jax.ShapeDtypeStruct((NBATCH, RANK), jnp.float32),
    mesh=_MESH2,
    compiler_params=_SC_PARAMS,
    scratch_types=[
        pltpu.VMEM((BGRP, GRP), jnp.int32),
        pltpu.VMEM((BPT, RANK), jnp.float32),
        pltpu.VMEM((BPT, RANK), jnp.float32),
        pltpu.VMEM((BPT, RANK), jnp.float32),
        pltpu.SemaphoreType.DMA,
    ],
)(_mean_gather_body)


def _mm_body(x_ref, v_ref, o_ref):
    o_ref[...] = jnp.dot(x_ref[...], v_ref[...],
                         preferred_element_type=jnp.float32)


def _expand(x, v):
    blk = 1024
    return pl.pallas_call(
        _mm_body,
        grid=(NBATCH // blk,),
        in_specs=[
            pl.BlockSpec((blk, RANK), lambda i: (i, 0)),
            pl.BlockSpec((RANK, EMB), lambda i: (0, 0)),
        ],
        out_specs=pl.BlockSpec((blk, EMB), lambda i: (i, 0)),
        out_shape=jax.ShapeDtypeStruct((NBATCH, EMB), jnp.float32),
    )(x, v)


def kernel(U, V, edge_index, batch):
    pad = EPAD - NEDGES
    src = jnp.concatenate([edge_index[0], jnp.zeros((pad,), jnp.int32)])
    dst = jnp.concatenate(
        [edge_index[1], jnp.full((pad,), DUMMY, jnp.int32)])
    srcg = src.reshape(EPAD // GRP, GRP)
    dstg = dst.reshape(EPAD // GRP, GRP)
    zeros = jnp.zeros((NROWS, RANK), jnp.float32)
    ones = jnp.ones((GRP, RANK), jnp.float32)

    acc, deg = _scatter_kernel(U, srcg, dstg, zeros, ones)
    outr = _mean_gather_kernel(acc, deg,
                               batch.reshape(NBATCH // GRP, GRP))
    return _expand(outr, V)


# speedup vs baseline: 1.6405x; 1.1060x over previous
import functools

import jax
import jax.numpy as jnp
from jax import lax
from jax.experimental import pallas as pl
from jax.experimental.pallas import tpu as pltpu
from jax.experimental.pallas import tpu_sc as plsc

NNODES = 10000
RANK = 16
WID2 = 2 * RANK
NEDGES = 320000
NBATCH = 8192
EMB = 128

NC = 2
NS = 16
NW = NC * NS

GRP = 128
EPT = 20480
NGRP = EPT // GRP
EPAD = EPT * NS
DUMMY = NNODES
NROWS = 10112
RPT = NROWS // NS

BPT = NBATCH // NW
BGRP = BPT // GRP

_MESH1 = plsc.VectorSubcoreMesh(
    core_axis_name="c", subcore_axis_name="s", num_cores=1, num_subcores=NS
)
_MESH2 = plsc.VectorSubcoreMesh(
    core_axis_name="c", subcore_axis_name="s", num_cores=NC, num_subcores=NS
)
_SC_PARAMS = pltpu.CompilerParams(use_tc_tiling_on_sc=False)

NBUF = 8
NCHUNK = NGRP // NBUF


URPT = 632
ULAST = NNODES - 15 * URPT


def _scatter_body(u_hbm, srcg, dstg, zeros_hbm, ones_hbm,
                  acc, deg,
                  acc_sh, deg_sh, u_sh, sidx, didx, rows, ones_v,
                  gsem, ssem, dsem):
    sid = lax.axis_index("s")
    r0 = sid * RPT

    pltpu.sync_copy(zeros_hbm.at[pl.ds(r0, RPT)], acc_sh.at[pl.ds(r0, RPT)])
    pltpu.sync_copy(zeros_hbm.at[pl.ds(r0, RPT)], deg_sh.at[pl.ds(r0, RPT)])

    @pl.when(sid < 15)
    def _():
        u0 = sid * URPT
        pltpu.sync_copy(u_hbm.at[pl.ds(u0, URPT)], u_sh.at[pl.ds(u0, URPT)])

    @pl.when(sid == 15)
    def _():
        u0 = 15 * URPT
        pltpu.sync_copy(u_hbm.at[pl.ds(u0, ULAST)],
                        u_sh.at[pl.ds(u0, ULAST)])

    pltpu.sync_copy(srcg.at[pl.ds(sid * NGRP, NGRP)], sidx)
    pltpu.sync_copy(dstg.at[pl.ds(sid * NGRP, NGRP)], didx)
    pltpu.sync_copy(ones_hbm, ones_v)

    plsc.subcore_barrier()

    def gfire(j0, h):
        return [pltpu.async_copy(u_sh.at[sidx.at[j0 + b]],
                                 rows.at[h * NBUF + b], gsem)
                for b in range(NBUF)]

    def dfire(j0):
        return [pltpu.async_copy(ones_v, deg_sh.at[didx.at[j0 + b]], dsem,
                                 add=True)
                for b in range(NBUF)]

    def sfire(j0, h):
        return [pltpu.async_copy(rows.at[h * NBUF + b],
                                 acc_sh.at[didx.at[j0 + b]], ssem, add=True)
                for b in range(NBUF)]

    def body(k, carry):
        j0 = 2 * k * NBUF
        j1 = j0 + NBUF
        gA = gfire(j0, 0)
        dA = dfire(j0)
        for d in gA:
            d.wait()
        sA = sfire(j0, 0)
        gB = gfire(j1, 1)
        dB = dfire(j1)
        for d in sA + dA + gB:
            d.wait()
        sB = sfire(j1, 1)
        for d in sB + dB:
            d.wait()
        return carry

    lax.fori_loop(0, NCHUNK // 2, body, 0)

    plsc.subcore_barrier()

    pltpu.sync_copy(acc_sh.at[pl.ds(r0, RPT)], acc.at[pl.ds(r0, RPT)])
    pltpu.sync_copy(deg_sh.at[pl.ds(r0, RPT)], deg.at[pl.ds(r0, RPT)])


_scatter_kernel = functools.partial(
    pl.kernel,
    out_type=[jax.ShapeDtypeStruct((NROWS, RANK), jnp.float32)] * 2,
    mesh=_MESH1,
    compiler_params=_SC_PARAMS,
    scratch_types=[
        pltpu.VMEM_SHARED((NROWS, RANK), jnp.float32),
        pltpu.VMEM_SHARED((NROWS, RANK), jnp.float32),
        pltpu.VMEM_SHARED((NNODES, RANK), jnp.float32),
        pltpu.VMEM((NGRP, GRP), jnp.int32),
        pltpu.VMEM((NGRP, GRP), jnp.int32),
        pltpu.VMEM((2 * NBUF, GRP, RANK), jnp.float32),
        pltpu.VMEM((GRP, RANK), jnp.float32),
        pltpu.SemaphoreType.DMA,
        pltpu.SemaphoreType.DMA,
        pltpu.SemaphoreType.DMA,
    ],
)(_scatter_body)


def _mean_gather_body(acc, deg, batchg, outr, bidx, a0, d0, ov, sem):
    cid = lax.axis_index("c")
    sid = lax.axis_index("s")
    wid = cid * NS + sid

    pltpu.sync_copy(batchg.at[pl.ds(wid * BGRP, BGRP)], bidx)
    ds = []
    for j in range(BGRP):
        sl = pl.ds(j * GRP, GRP)
        ds.append(pltpu.async_copy(acc.at[bidx.at[j]], a0.at[sl], sem))
        ds.append(pltpu.async_copy(deg.at[bidx.at[j]], d0.at[sl], sem))
    for d in ds:
        d.wait()

    def body(i, carry):
        ov[i] = a0[i] / jnp.maximum(d0[i], 1.0)
        return carry

    lax.fori_loop(0, BPT, body, 0)

    pltpu.sync_copy(ov, outr.at[pl.ds(wid * BPT, BPT)])


_mean_gather_kernel = functools.partial(
    pl.kernel,
    out_type=jax.ShapeDtypeStruct((NBATCH, RANK), jnp.float32),
    mesh=_MESH2,
    compiler_params=_SC_PARAMS,
    scratch_types=[
        pltpu.VMEM((BGRP, GRP), jnp.int32),
        pltpu.VMEM((BPT, RANK), jnp.float32),
        pltpu.VMEM((BPT, RANK), jnp.float32),
        pltpu.VMEM((BPT, RANK), jnp.float32),
        pltpu.SemaphoreType.DMA,
    ],
)(_mean_gather_body)


def _mm_body(x_ref, v_ref, o_ref):
    o_ref[...] = jnp.dot(x_ref[...], v_ref[...],
                         preferred_element_type=jnp.float32)


def _expand(x, v):
    blk = 1024
    return pl.pallas_call(
        _mm_body,
        grid=(NBATCH // blk,),
        in_specs=[
            pl.BlockSpec((blk, RANK), lambda i: (i, 0)),
            pl.BlockSpec((RANK, EMB), lambda i: (0, 0)),
        ],
        out_specs=pl.BlockSpec((blk, EMB), lambda i: (i, 0)),
        out_shape=jax.ShapeDtypeStruct((NBATCH, EMB), jnp.float32),
    )(x, v)


def kernel(U, V, edge_index, batch):
    pad = EPAD - NEDGES
    src = jnp.concatenate([edge_index[0], jnp.zeros((pad,), jnp.int32)])
    dst = jnp.concatenate(
        [edge_index[1], jnp.full((pad,), DUMMY, jnp.int32)])
    srcg = src.reshape(EPAD // GRP, GRP)
    dstg = dst.reshape(EPAD // GRP, GRP)
    zeros = jnp.zeros((NROWS, RANK), jnp.float32)
    ones = jnp.ones((GRP, RANK), jnp.float32)

    acc, deg = _scatter_kernel(U, srcg, dstg, zeros, ones)
    outr = _mean_gather_kernel(acc, deg,
                               batch.reshape(NBATCH // GRP, GRP))
    return _expand(outr, V)


# speedup vs baseline: 1.6646x; 1.0147x over previous
import functools

import jax
import jax.numpy as jnp
from jax import lax
from jax.experimental import pallas as pl
from jax.experimental.pallas import tpu as pltpu
from jax.experimental.pallas import tpu_sc as plsc

NNODES = 10000
RANK = 16
WID2 = 2 * RANK
NEDGES = 320000
NBATCH = 8192
EMB = 128

NC = 2
NS = 16
NW = NC * NS

GRP = 128
EPT = 20480
NGRP = EPT // GRP
EPAD = EPT * NS
DUMMY = NNODES
NROWS = 10112
RPT = NROWS // NS

BPT = NBATCH // NW
BGRP = BPT // GRP

_MESH1 = plsc.VectorSubcoreMesh(
    core_axis_name="c", subcore_axis_name="s", num_cores=1, num_subcores=NS
)
_MESH2 = plsc.VectorSubcoreMesh(
    core_axis_name="c", subcore_axis_name="s", num_cores=NC, num_subcores=NS
)
_SC_PARAMS = pltpu.CompilerParams(use_tc_tiling_on_sc=False)

NBUF = 8
NCHUNK = NGRP // NBUF


URPT = 632
ULAST = NNODES - 15 * URPT


NGREAL = NEDGES // GRP
NGR15 = NGREAL - 15 * NGRP
NGPAD = NGRP - NGR15


def _scatter_body(u_hbm, srcg, dstg, srcp, dstp, zeros_hbm, ones_hbm,
                  acc, deg,
                  acc_sh, deg_sh, u_sh, sidx, didx, rows, ones_v,
                  gsem, ssem, dsem):
    sid = lax.axis_index("s")
    r0 = sid * RPT

    pltpu.sync_copy(zeros_hbm.at[pl.ds(r0, RPT)], acc_sh.at[pl.ds(r0, RPT)])
    pltpu.sync_copy(zeros_hbm.at[pl.ds(r0, RPT)], deg_sh.at[pl.ds(r0, RPT)])

    @pl.when(sid < 15)
    def _():
        u0 = sid * URPT
        pltpu.sync_copy(u_hbm.at[pl.ds(u0, URPT)], u_sh.at[pl.ds(u0, URPT)])

    @pl.when(sid == 15)
    def _():
        u0 = 15 * URPT
        pltpu.sync_copy(u_hbm.at[pl.ds(u0, ULAST)],
                        u_sh.at[pl.ds(u0, ULAST)])

    @pl.when(sid < 15)
    def _():
        pltpu.sync_copy(srcg.at[pl.ds(sid * NGRP, NGRP)], sidx)
        pltpu.sync_copy(dstg.at[pl.ds(sid * NGRP, NGRP)], didx)

    @pl.when(sid == 15)
    def _():
        pltpu.sync_copy(srcg.at[pl.ds(15 * NGRP, NGR15)],
                        sidx.at[pl.ds(0, NGR15)])
        pltpu.sync_copy(srcp, sidx.at[pl.ds(NGR15, NGPAD)])
        pltpu.sync_copy(dstg.at[pl.ds(15 * NGRP, NGR15)],
                        didx.at[pl.ds(0, NGR15)])
        pltpu.sync_copy(dstp, didx.at[pl.ds(NGR15, NGPAD)])

    pltpu.sync_copy(ones_hbm, ones_v)

    plsc.subcore_barrier()

    def gfire(j0, h):
        return [pltpu.async_copy(u_sh.at[sidx.at[j0 + b]],
                                 rows.at[h * NBUF + b], gsem)
                for b in range(NBUF)]

    def dfire(j0):
        return [pltpu.async_copy(ones_v, deg_sh.at[didx.at[j0 + b]], dsem,
                                 add=True)
                for b in range(NBUF)]

    def sfire(j0, h):
        return [pltpu.async_copy(rows.at[h * NBUF + b],
                                 acc_sh.at[didx.at[j0 + b]], ssem, add=True)
                for b in range(NBUF)]

    def body(k, carry):
        j0 = 2 * k * NBUF
        j1 = j0 + NBUF
        gA = gfire(j0, 0)
        dA = dfire(j0)
        for d in gA:
            d.wait()
        sA = sfire(j0, 0)
        gB = gfire(j1, 1)
        dB = dfire(j1)
        for d in sA + dA + gB:
            d.wait()
        sB = sfire(j1, 1)
        for d in sB + dB:
            d.wait()
        return carry

    lax.fori_loop(0, NCHUNK // 2, body, 0)

    plsc.subcore_barrier()

    pltpu.sync_copy(acc_sh.at[pl.ds(r0, RPT)], acc.at[pl.ds(r0, RPT)])
    pltpu.sync_copy(deg_sh.at[pl.ds(r0, RPT)], deg.at[pl.ds(r0, RPT)])


_scatter_kernel = functools.partial(
    pl.kernel,
    out_type=[jax.ShapeDtypeStruct((NROWS, RANK), jnp.float32)] * 2,
    mesh=_MESH1,
    compiler_params=_SC_PARAMS,
    scratch_types=[
        pltpu.VMEM_SHARED((NROWS, RANK), jnp.float32),
        pltpu.VMEM_SHARED((NROWS, RANK), jnp.float32),
        pltpu.VMEM_SHARED((NNODES, RANK), jnp.float32),
        pltpu.VMEM((NGRP, GRP), jnp.int32),
        pltpu.VMEM((NGRP, GRP), jnp.int32),
        pltpu.VMEM((2 * NBUF, GRP, RANK), jnp.float32),
        pltpu.VMEM((GRP, RANK), jnp.float32),
        pltpu.SemaphoreType.DMA,
        pltpu.SemaphoreType.DMA,
        pltpu.SemaphoreType.DMA,
    ],
)(_scatter_body)


def _mean_gather_body(acc, deg, batchg, outr, bidx, a0, d0, ov, sem):
    cid = lax.axis_index("c")
    sid = lax.axis_index("s")
    wid = cid * NS + sid

    pltpu.sync_copy(batchg.at[pl.ds(wid * BGRP, BGRP)], bidx)
    ds = []
    for j in range(BGRP):
        sl = pl.ds(j * GRP, GRP)
        ds.append(pltpu.async_copy(acc.at[bidx.at[j]], a0.at[sl], sem))
        ds.append(pltpu.async_copy(deg.at[bidx.at[j]], d0.at[sl], sem))
    for d in ds:
        d.wait()

    def body(i, carry):
        ov[i] = a0[i] / jnp.maximum(d0[i], 1.0)
        return carry

    lax.fori_loop(0, BPT, body, 0)

    pltpu.sync_copy(ov, outr.at[pl.ds(wid * BPT, BPT)])


_mean_gather_kernel = functools.partial(
    pl.kernel,
    out_type=jax.ShapeDtypeStruct((NBATCH, RANK), jnp.float32),
    mesh=_MESH2,
    compiler_params=_SC_PARAMS,
    scratch_types=[
        pltpu.VMEM((BGRP, GRP), jnp.int32),
        pltpu.VMEM((BPT, RANK), jnp.float32),
        pltpu.VMEM((BPT, RANK), jnp.float32),
        pltpu.VMEM((BPT, RANK), jnp.float32),
        pltpu.SemaphoreType.DMA,
    ],
)(_mean_gather_body)


def _mm_body(x_ref, v_ref, o_ref):
    o_ref[...] = jnp.dot(x_ref[...], v_ref[...],
                         preferred_element_type=jnp.float32)


def _expand(x, v):
    blk = 1024
    return pl.pallas_call(
        _mm_body,
        grid=(NBATCH // blk,),
        in_specs=[
            pl.BlockSpec((blk, RANK), lambda i: (i, 0)),
            pl.BlockSpec((RANK, EMB), lambda i: (0, 0)),
        ],
        out_specs=pl.BlockSpec((blk, EMB), lambda i: (i, 0)),
        out_shape=jax.ShapeDtypeStruct((NBATCH, EMB), jnp.float32),
    )(x, v)


def kernel(U, V, edge_index, batch):
    srcg = edge_index[0].reshape(NGREAL, GRP)
    dstg = edge_index[1].reshape(NGREAL, GRP)
    srcp = jnp.zeros((NGPAD, GRP), jnp.int32)
    dstp = jnp.full((NGPAD, GRP), DUMMY, jnp.int32)
    zeros = jnp.zeros((NROWS, RANK), jnp.float32)
    ones = jnp.ones((GRP, RANK), jnp.float32)

    acc, deg = _scatter_kernel(U, srcg, dstg, srcp, dstp, zeros, ones)
    outr = _mean_gather_kernel(acc, deg,
                               batch.reshape(NBATCH // GRP, GRP))
    return _expand(outr, V)


# speedup vs baseline: 1.8464x; 1.1092x over previous
import functools

import jax
import jax.numpy as jnp
from jax import lax
from jax.experimental import pallas as pl
from jax.experimental.pallas import tpu as pltpu
from jax.experimental.pallas import tpu_sc as plsc

NNODES = 10000
RANK = 16
WID2 = 2 * RANK
NEDGES = 320000
NBATCH = 8192
EMB = 128

NC = 2
NS = 16
NW = NC * NS

GRP = 128
EPT = 20480
NGRP = EPT // GRP
EPAD = EPT * NS
DUMMY = NNODES
NROWS = 10112
RPT = NROWS // NS

BPT = NBATCH // NW
BGRP = BPT // GRP

_MESH1 = plsc.VectorSubcoreMesh(
    core_axis_name="c", subcore_axis_name="s", num_cores=1, num_subcores=NS
)
_MESH2 = plsc.VectorSubcoreMesh(
    core_axis_name="c", subcore_axis_name="s", num_cores=NC, num_subcores=NS
)
_SC_PARAMS = pltpu.CompilerParams(use_tc_tiling_on_sc=False,
                                  needs_layout_passes=False)

DROWS = 640
DRPT = DROWS // NS
NIDENT = DROWS // GRP

NBUF = 8
NCHUNK = NGRP // NBUF


URPT = 632
ULAST = NNODES - 15 * URPT


NGREAL = NEDGES // GRP
NGR15 = NGREAL - 15 * NGRP
NGPAD = NGRP - NGR15


def _scatter_body(u_hbm, srcg, dstg, srcp, dstp, zeros_hbm, ident_hbm,
                  acc, deg2,
                  acc_sh, deg_sh2, u_sh, sidx, didx, rows, degt, identv,
                  gsem, ssem, msem):
    sid = lax.axis_index("s")
    r0 = sid * RPT

    pltpu.sync_copy(zeros_hbm.at[pl.ds(r0, RPT)], acc_sh.at[pl.ds(r0, RPT)])
    pltpu.sync_copy(zeros_hbm.at[pl.ds(0, DRPT)],
                    deg_sh2.at[pl.ds(sid * DRPT, DRPT)])
    pltpu.sync_copy(ident_hbm, identv)

    def zrow(i, c):
        degt[i] = jnp.zeros((RANK,), jnp.float32)
        return c
    lax.fori_loop(0, DROWS, zrow, 0)

    @pl.when(sid < 15)
    def _():
        u0 = sid * URPT
        pltpu.sync_copy(u_hbm.at[pl.ds(u0, URPT)], u_sh.at[pl.ds(u0, URPT)])

    @pl.when(sid == 15)
    def _():
        u0 = 15 * URPT
        pltpu.sync_copy(u_hbm.at[pl.ds(u0, ULAST)],
                        u_sh.at[pl.ds(u0, ULAST)])

    @pl.when(sid < 15)
    def _():
        pltpu.sync_copy(srcg.at[pl.ds(sid * NGRP, NGRP)], sidx)
        pltpu.sync_copy(dstg.at[pl.ds(sid * NGRP, NGRP)], didx)

    @pl.when(sid == 15)
    def _():
        pltpu.sync_copy(srcg.at[pl.ds(15 * NGRP, NGR15)],
                        sidx.at[pl.ds(0, NGR15)])
        pltpu.sync_copy(srcp, sidx.at[pl.ds(NGR15, NGPAD)])
        pltpu.sync_copy(dstg.at[pl.ds(15 * NGRP, NGR15)],
                        didx.at[pl.ds(0, NGR15)])
        pltpu.sync_copy(dstp, didx.at[pl.ds(NGR15, NGPAD)])

    plsc.subcore_barrier()

    ones16 = jnp.ones((16,), jnp.float32)

    def gfire(j0, h):
        return [pltpu.async_copy(u_sh.at[sidx.at[j0 + b]],
                                 rows.at[h * NBUF + b], gsem)
                for b in range(NBUF)]

    def sfire(j0, h):
        return [pltpu.async_copy(rows.at[h * NBUF + b],
                                 acc_sh.at[didx.at[j0 + b]], ssem, add=True)
                for b in range(NBUF)]

    def dcount(j0):
        for b in range(NBUF):
            for v in range(GRP // 16):
                dv = didx[j0 + b, pl.ds(v * 16, 16)]
                plsc.addupdate_scatter(degt, [dv >> 4, dv & 15], ones16)

    def body(k, carry):
        j0 = 2 * k * NBUF
        j1 = j0 + NBUF
        gA = gfire(j0, 0)
        for d in gA:
            d.wait()
        sA = sfire(j0, 0)
        gB = gfire(j1, 1)
        dcount(j0)
        for d in sA + gB:
            d.wait()
        sB = sfire(j1, 1)
        dcount(j1)
        for d in sB:
            d.wait()
        return carry

    lax.fori_loop(0, NCHUNK // 2, body, 0)

    mds = [pltpu.async_copy(degt.at[pl.ds(k * GRP, GRP)],
                            deg_sh2.at[identv.at[k]], msem, add=True)
           for k in range(NIDENT)]
    for d in mds:
        d.wait()

    plsc.subcore_barrier()

    pltpu.sync_copy(acc_sh.at[pl.ds(r0, RPT)], acc.at[pl.ds(r0, RPT)])
    pltpu.sync_copy(deg_sh2.at[pl.ds(sid * DRPT, DRPT)],
                    deg2.at[pl.ds(sid * DRPT, DRPT)])


_scatter_kernel = functools.partial(
    pl.kernel,
    out_type=[jax.ShapeDtypeStruct((NROWS, RANK), jnp.float32),
              jax.ShapeDtypeStruct((DROWS, RANK), jnp.float32)],
    mesh=_MESH1,
    compiler_params=_SC_PARAMS,
    scratch_types=[
        pltpu.VMEM_SHARED((NROWS, RANK), jnp.float32),
        pltpu.VMEM_SHARED((DROWS, RANK), jnp.float32),
        pltpu.VMEM_SHARED((NNODES, RANK), jnp.float32),
        pltpu.VMEM((NGRP, GRP), jnp.int32),
        pltpu.VMEM((NGRP, GRP), jnp.int32),
        pltpu.VMEM((2 * NBUF, GRP, RANK), jnp.float32),
        pltpu.VMEM((DROWS, RANK), jnp.float32),
        pltpu.VMEM((NIDENT, GRP), jnp.int32),
        pltpu.SemaphoreType.DMA,
        pltpu.SemaphoreType.DMA,
        pltpu.SemaphoreType.DMA,
    ],
)(_scatter_body)


def _mean_gather_body(acc, deg2, batchg, batchr, outr, bidx, ridx, a0, d0,
                      ov, dtmp, sem):
    cid = lax.axis_index("c")
    sid = lax.axis_index("s")
    wid = cid * NS + sid

    pltpu.sync_copy(batchg.at[pl.ds(wid * BGRP, BGRP)], bidx)
    pltpu.sync_copy(batchr.at[pl.ds(wid * BGRP, BGRP)], ridx)
    ds = []
    for j in range(BGRP):
        sl = pl.ds(j * GRP, GRP)
        ds.append(pltpu.async_copy(acc.at[bidx.at[j]], a0.at[sl], sem))
        ds.append(pltpu.async_copy(deg2.at[ridx.at[j]], d0.at[sl], sem))
    for d in ds:
        d.wait()

    iota16 = jnp.arange(16, dtype=jnp.int32)

    def body(v, carry):
        i0 = v * 16
        j = v // 8
        bv = bidx[j, pl.ds((v % 8) * 16, 16)]
        dvals = plsc.load_gather(d0, [i0 + iota16, bv & 15])
        dmax = jnp.maximum(dvals, 1.0)
        dnums = lax.GatherDimensionNumbers(
            offset_dims=(), collapsed_slice_dims=(0,), start_index_map=(0,))
        for k in range(16):
            dk = lax.gather(dmax, jnp.full((16, 1), k, jnp.int32), dnums,
                            (1,), mode=lax.GatherScatterMode.PROMISE_IN_BOUNDS)
            ov[i0 + k] = a0[i0 + k] / dk
        return carry

    lax.fori_loop(0, BPT // 16, body, 0)

    pltpu.sync_copy(ov, outr.at[pl.ds(wid * BPT, BPT)])


_mean_gather_kernel = functools.partial(
    pl.kernel,
    out_type=jax.ShapeDtypeStruct((NBATCH, RANK), jnp.float32),
    mesh=_MESH2,
    compiler_params=_SC_PARAMS,
    scratch_types=[
        pltpu.VMEM((BGRP, GRP), jnp.int32),
        pltpu.VMEM((BGRP, GRP), jnp.int32),
        pltpu.VMEM((BPT, RANK), jnp.float32),
        pltpu.VMEM((BPT, RANK), jnp.float32),
        pltpu.VMEM((BPT, RANK), jnp.float32),
        pltpu.VMEM((16,), jnp.float32),
        pltpu.SemaphoreType.DMA,
    ],
)(_mean_gather_body)


def _mm_body(x_ref, v_ref, o_ref):
    o_ref[...] = jnp.dot(x_ref[...], v_ref[...],
                         preferred_element_type=jnp.float32)


def _expand(x, v):
    blk = 1024
    return pl.pallas_call(
        _mm_body,
        grid=(NBATCH // blk,),
        in_specs=[
            pl.BlockSpec((blk, RANK), lambda i: (i, 0)),
            pl.BlockSpec((RANK, EMB), lambda i: (0, 0)),
        ],
        out_specs=pl.BlockSpec((blk, EMB), lambda i: (i, 0)),
        out_shape=jax.ShapeDtypeStruct((NBATCH, EMB), jnp.float32),
    )(x, v)


def kernel(U, V, edge_index, batch):
    srcg = edge_index[0].reshape(NGREAL, GRP)
    dstg = edge_index[1].reshape(NGREAL, GRP)
    srcp = jnp.zeros((NGPAD, GRP), jnp.int32)
    dstp = jnp.full((NGPAD, GRP), DUMMY, jnp.int32)
    zeros = jnp.zeros((NROWS, RANK), jnp.float32)
    ident = jnp.arange(DROWS, dtype=jnp.int32).reshape(NIDENT, GRP)

    acc, deg2 = _scatter_kernel(U, srcg, dstg, srcp, dstp, zeros, ident)
    outr = _mean_gather_kernel(acc, deg2,
                               batch.reshape(NBATCH // GRP, GRP),
                               (batch >> 4).reshape(NBATCH // GRP, GRP))
    return _expand(outr, V)
